# baseline (speedup 1.0000x reference)
import jax
import jax.numpy as jnp
from jax import lax
from jax.experimental import pallas as pl
from jax.experimental.pallas import tpu as pltpu
from jax.experimental.pallas import tpu_sc as plsc

NTASKS = 1000000
FRONTIER = 10
BATCH = 16384

_NC = 2
_NS = 16
_NW = _NC * _NS
_PER_W = BATCH // _NW
_CHUNK = 128
_NCHUNK = _PER_W // _CHUNK
_L = 16


def _sc_gather(wflat_hbm, n_hbm, i2d_hbm, w_out, n_out,
               idx_v, idxw_v, wt_v, nv_v, sem_w, sem_n):
    wid = lax.axis_index("s") * _NC + lax.axis_index("c")
    base = wid * _PER_W
    pltpu.sync_copy(i2d_hbm.at[pl.ds(wid * _NCHUNK, _NCHUNK)], idx_v)
    for c in range(_NCHUNK):
        for k in range(_CHUNK // _L):
            sl = pl.ds(k * _L, _L)
            ten = idx_v[c, sl] * FRONTIER
            for j in range(FRONTIER):
                idxw_v[j, c, sl] = ten + j
    copies = []
    for c in range(_NCHUNK):
        copies.append(pltpu.async_copy(
            n_hbm.at[idx_v.at[c]], nv_v.at[pl.ds(c * _CHUNK, _CHUNK)],
            sem_n))
        for j in range(FRONTIER):
            copies.append(pltpu.async_copy(
                wflat_hbm.at[idxw_v.at[j, c]],
                wt_v.at[j, pl.ds(c * _CHUNK, _CHUNK)], sem_w))
    for cp in copies:
        cp.wait()
    pltpu.sync_copy(wt_v, w_out.at[:, pl.ds(base, _PER_W)])
    pltpu.sync_copy(nv_v, n_out.at[pl.ds(base, _PER_W)])


def _tc_body(w_ref, n_ref, lik_ref, o_ref):
    w = w_ref[...]
    m1 = jnp.max(w, axis=0, keepdims=True)
    lse_w = m1 + jnp.log(jnp.sum(jnp.exp(w - m1), axis=0, keepdims=True))
    logprobs = w - lse_w
    comp = lax.broadcasted_iota(jnp.int32, (FRONTIER, 1), 0).astype(
        jnp.float32)
    mask = n_ref[...] > comp
    scores = jnp.where(mask, logprobs + lik_ref[...], jnp.float32(-1e30))
    m2 = jnp.max(scores, axis=0, keepdims=True)
    o_ref[...] = m2 + jnp.log(
        jnp.sum(jnp.exp(scores - m2), axis=0, keepdims=True))


@jax.jit
def kernel(mixtureWeights, nMixtureComponents, likelihoods, i):
    i2d = i.astype(jnp.int32).reshape(BATCH // _CHUNK, _CHUNK)
    wflat = mixtureWeights.reshape(NTASKS * FRONTIER)

    mesh = plsc.VectorSubcoreMesh(core_axis_name="c", subcore_axis_name="s")
    w_g, n_g = pl.kernel(
        _sc_gather,
        out_type=(
            jax.ShapeDtypeStruct((FRONTIER, BATCH), jnp.float32),
            jax.ShapeDtypeStruct((BATCH,), jnp.float32),
        ),
        mesh=mesh,
        compiler_params=pltpu.CompilerParams(use_tc_tiling_on_sc=False),
        scratch_types=[
            pltpu.VMEM((_NCHUNK, _CHUNK), jnp.int32),
            pltpu.VMEM((FRONTIER, _NCHUNK, _CHUNK), jnp.int32),
            pltpu.VMEM((FRONTIER, _PER_W), jnp.float32),
            pltpu.VMEM((_PER_W,), jnp.float32),
            pltpu.SemaphoreType.DMA,
            pltpu.SemaphoreType.DMA,
        ],
    )(wflat, nMixtureComponents, i2d)

    out = pl.pallas_call(
        _tc_body,
        out_shape=jax.ShapeDtypeStruct((1, BATCH), jnp.float32),
    )(w_g, n_g.reshape(1, BATCH), likelihoods.T)
    return out.reshape(BATCH)
